# baseline (speedup 1.0000x reference)
import functools

import jax
import jax.numpy as jnp
from jax import lax
from jax.experimental import pallas as pl
from jax.experimental.pallas import tpu as pltpu
from jax.experimental.pallas import tpu_sc as plsc

_NNODE = 10000
_NATTRI = 6000
_NFEAT = 128
_NHID = 64
_NHID2 = 32
_E = 512000
_N = _NNODE + _NATTRI

_NW = 32
_CH = 128
_EPW = _E // _NW
_NCHUNK = _EPW // _CH
_ZR = _N // 16
_BM = 2000
_NBUF = 5
_PFI = 4
_PF = 2


def _make_spmm(D):
    mesh = plsc.VectorSubcoreMesh(core_axis_name="c", subcore_axis_name="s")
    NG = _NCHUNK // 5

    @functools.partial(
        pl.kernel,
        out_type=jax.ShapeDtypeStruct((2, _N, D), jnp.float32),
        mesh=mesh,
        compiler_params=pltpu.CompilerParams(use_tc_tiling_on_sc=False),
        scratch_types=[
            pltpu.VMEM((4, 5, 2, _CH), jnp.int32),
            pltpu.VMEM((4, 5, _CH), jnp.float32),
            pltpu.VMEM((5, _CH, D), jnp.float32),
            pltpu.VMEM_SHARED((_N, D), jnp.float32),
            pltpu.SemaphoreType.DMA((4,)),
            pltpu.SemaphoreType.DMA((5,)),
            pltpu.SemaphoreType.DMA((5,)),
        ],
    )
    def spmm(edata_hbm, w_hbm, sup_hbm, out_hbm,
             ebuf, wbuf, rows_v, acc, isem, gsem, ssem):
        cid = lax.axis_index("c")
        sid = lax.axis_index("s")
        wid = sid * 2 + cid

        zvec = jnp.zeros((16,), jnp.float32)

        def zrow(e, carry):
            for q in range(D // 16):
                rows_v[0, e, pl.ds(q * 16, 16)] = zvec
            return carry

        lax.fori_loop(0, _CH, zrow, 0)
        for k in range(8):
            pltpu.sync_copy(rows_v.at[0].at[pl.ds(0, _ZR // 8)],
                            acc.at[pl.ds(sid * _ZR + k * (_ZR // 8), _ZR // 8)])
        plsc.subcore_barrier()

        def idx_start(grp, slot):
            pltpu.async_copy(edata_hbm.at[wid, pl.ds(grp * 5, 5)],
                             ebuf.at[slot], isem.at[slot])
            pltpu.async_copy(w_hbm.at[wid, pl.ds(grp * 5, 5)],
                             wbuf.at[slot], isem.at[slot])

        def idx_wait(grp, slot):
            pltpu.make_async_copy(edata_hbm.at[wid, pl.ds(grp * 5, 5)],
                                  ebuf.at[slot], isem.at[slot]).wait()
            pltpu.make_async_copy(w_hbm.at[wid, pl.ds(grp * 5, 5)],
                                  wbuf.at[slot], isem.at[slot]).wait()

        def gather_start(gslot, pi, rslot):
            pltpu.async_copy(sup_hbm.at[ebuf.at[gslot, pi, 0]],
                             rows_v.at[rslot], gsem.at[rslot])

        def gather_wait(gslot, pi, rslot):
            pltpu.make_async_copy(sup_hbm.at[ebuf.at[gslot, pi, 0]],
                                  rows_v.at[rslot], gsem.at[rslot]).wait()

        def scatter_start(gslot, pi, rslot):
            pltpu.async_copy(rows_v.at[rslot], acc.at[ebuf.at[gslot, pi, 1]],
                             ssem.at[rslot], add=True)

        def scatter_wait(gslot, pi, rslot):
            pltpu.make_async_copy(rows_v.at[rslot], acc.at[ebuf.at[gslot, pi, 1]],
                                  ssem.at[rslot]).wait()

        for grp in range(3):
            idx_start(grp, grp)
        idx_wait(0, 0)
        gather_start(0, 0, 0)
        gather_start(0, 1, 1)

        def group(g, carry):
            gslot = g % 4
            gslot1 = (g + 1) % 4
            for k in range(5):
                s2 = (k + 2) % 5
                if k < 3:
                    @pl.when(g >= 1)
                    def _():
                        scatter_wait(gslot, k + 2, s2)

                    gather_start(gslot, k + 2, s2)
                else:
                    scatter_wait(gslot, k - 3, s2)
                    if k == 3:
                        @pl.when(g < NG - 1)
                        def _():
                            idx_wait(g + 1, gslot1)
                            gather_start(gslot1, 0, s2)

                        @pl.when(g < NG - 3)
                        def _():
                            idx_start(g + 3, (g + 3) % 4)
                    else:
                        @pl.when(g < NG - 1)
                        def _():
                            gather_start(gslot1, 1, s2)

                gather_wait(gslot, k, k)

                def scale(g2, c2):
                    w16 = wbuf[gslot, k, pl.ds(g2 * 16, 16)]
                    for l in range(16):
                        e = g2 * 16 + l
                        wsc = w16[l]
                        for q in range(D // 16):
                            sl = pl.ds(q * 16, 16)
                            rows_v[k, e, sl] = rows_v[k, e, sl] * wsc
                    return c2

                lax.fori_loop(0, _CH // 16, scale, 0)
                scatter_start(gslot, k, k)
            return carry

        lax.fori_loop(0, NG, group, 0)
        last = (NG - 1) % 4
        for k in range(2, 5):
            scatter_wait(last, k, k)

        plsc.subcore_barrier()
        pltpu.sync_copy(acc.at[pl.ds(sid * _ZR, _ZR)],
                        out_hbm.at[cid, pl.ds(sid * _ZR, _ZR)])

    return spmm


def _mm1(x, W1):
    def body(x_ref, w_ref, o_ref):
        o_ref[...] = jnp.dot(x_ref[...], w_ref[...],
                             preferred_element_type=jnp.float32)

    return pl.pallas_call(
        body,
        grid=(_N // _BM,),
        in_specs=[pl.BlockSpec((_BM, _NFEAT), lambda i: (i, 0)),
                  pl.BlockSpec((_NFEAT, _NHID), lambda i: (0, 0))],
        out_specs=pl.BlockSpec((_BM, _NHID), lambda i: (i, 0)),
        out_shape=jax.ShapeDtypeStruct((_N, _NHID), jnp.float32),
    )(x, W1)


def _fuse1(parts, b1, W2):
    def body(p_ref, b_ref, w_ref, o_ref):
        h = jnp.maximum(p_ref[0] + p_ref[1] + b_ref[...], 0.0)
        o_ref[...] = jnp.dot(h, w_ref[...], preferred_element_type=jnp.float32)

    return pl.pallas_call(
        body,
        grid=(_N // _BM,),
        in_specs=[pl.BlockSpec((2, _BM, _NHID), lambda i: (0, i, 0)),
                  pl.BlockSpec((1, _NHID), lambda i: (0, 0)),
                  pl.BlockSpec((_NHID, _NHID2), lambda i: (0, 0))],
        out_specs=pl.BlockSpec((_BM, _NHID2), lambda i: (i, 0)),
        out_shape=jax.ShapeDtypeStruct((_N, _NHID2), jnp.float32),
    )(parts, b1.reshape(1, _NHID), W2)


def _fuse2(parts, b2):
    def body(p_ref, b_ref, o_ref):
        o_ref[...] = jnp.maximum(p_ref[0] + p_ref[1] + b_ref[...], 0.0)

    return pl.pallas_call(
        body,
        grid=(_N // _BM,),
        in_specs=[pl.BlockSpec((2, _BM, _NHID2), lambda i: (0, i, 0)),
                  pl.BlockSpec((1, _NHID2), lambda i: (0, 0))],
        out_specs=pl.BlockSpec((_BM, _NHID2), lambda i: (i, 0)),
        out_shape=jax.ShapeDtypeStruct((_N, _NHID2), jnp.float32),
    )(parts, b2.reshape(1, _NHID2))


def kernel(edge_index, edge_weight, emb_node, emb_attri, W1, b1, W2, b2):
    dst = edge_index[0].astype(jnp.int32).reshape(_NW, _NCHUNK, _CH)
    src = edge_index[1].astype(jnp.int32).reshape(_NW, _NCHUNK, _CH)
    w = edge_weight.astype(jnp.float32).reshape(_NW, _NCHUNK, _CH)
    edata = jnp.stack([src, dst], axis=2)

    x = jnp.concatenate([emb_node, emb_attri], axis=0)
    sup1 = _mm1(x, W1)
    part1 = _make_spmm(_NHID)(edata, w, sup1)
    sup2 = _fuse1(part1, b1, W2)
    part2 = _make_spmm(_NHID2)(edata, w, sup2)
    return _fuse2(part2, b2)


# speedup vs baseline: 1.2165x; 1.2165x over previous
import functools

import jax
import jax.numpy as jnp
from jax import lax
from jax.experimental import pallas as pl
from jax.experimental.pallas import tpu as pltpu
from jax.experimental.pallas import tpu_sc as plsc

_NNODE = 10000
_NATTRI = 6000
_NFEAT = 128
_NHID = 64
_NHID2 = 32
_E = 512000
_N = _NNODE + _NATTRI

_NW = 32
_CH = 128
_EPW = _E // _NW
_NCHUNK = _EPW // _CH
_ZR = _N // 16
_BM = 2000
_NBUF = 5
_PFI = 4
_PF = 2


def _make_spmm(D):
    mesh = plsc.VectorSubcoreMesh(core_axis_name="c", subcore_axis_name="s")

    @functools.partial(
        pl.kernel,
        out_type=jax.ShapeDtypeStruct((2, _N, D), jnp.float32),
        mesh=mesh,
        compiler_params=pltpu.CompilerParams(use_tc_tiling_on_sc=False),
        scratch_types=[
            pltpu.VMEM((_NBUF, 2, _CH), jnp.int32),
            pltpu.VMEM((_NBUF, _CH), jnp.float32),
            pltpu.VMEM((_NBUF, _CH, D), jnp.float32),
            pltpu.VMEM_SHARED((_N, D), jnp.float32),
            pltpu.SemaphoreType.DMA((_NBUF,)),
            pltpu.SemaphoreType.DMA((_NBUF,)),
            pltpu.SemaphoreType.DMA((_NBUF,)),
        ],
    )
    def spmm(edata_hbm, w_hbm, sup_hbm, out_hbm,
             ebuf, wbuf, rows_v, acc, isem, gsem, ssem):
        cid = lax.axis_index("c")
        sid = lax.axis_index("s")
        wid = sid * 2 + cid

        zvec = jnp.zeros((16,), jnp.float32)

        def zrow(e, carry):
            for q in range(D // 16):
                rows_v[0, e, pl.ds(q * 16, 16)] = zvec
            return carry

        lax.fori_loop(0, _CH, zrow, 0)
        for k in range(8):
            pltpu.sync_copy(rows_v.at[0].at[pl.ds(0, _ZR // 8)],
                            acc.at[pl.ds(sid * _ZR + k * (_ZR // 8), _ZR // 8)])
        plsc.subcore_barrier()

        def idx_start(f, bf):
            pltpu.async_copy(edata_hbm.at[wid, f], ebuf.at[bf], isem.at[bf])
            pltpu.async_copy(w_hbm.at[wid, f], wbuf.at[bf], isem.at[bf])

        def idx_wait(f, bf):
            pltpu.make_async_copy(edata_hbm.at[wid, f], ebuf.at[bf],
                                  isem.at[bf]).wait()
            pltpu.make_async_copy(w_hbm.at[wid, f], wbuf.at[bf],
                                  isem.at[bf]).wait()

        def gather_start(f, bf):
            pltpu.async_copy(sup_hbm.at[ebuf.at[bf, 0]], rows_v.at[bf],
                             gsem.at[bf])

        def gather_wait(f, bf):
            pltpu.make_async_copy(sup_hbm.at[ebuf.at[bf, 0]], rows_v.at[bf],
                                  gsem.at[bf]).wait()

        def scatter_start(f, bf):
            pltpu.async_copy(rows_v.at[bf], acc.at[ebuf.at[bf, 1]],
                             ssem.at[bf], add=True)

        def scatter_wait(f, bf):
            pltpu.make_async_copy(rows_v.at[bf], acc.at[ebuf.at[bf, 1]],
                                  ssem.at[bf]).wait()

        for f in range(_PFI):
            idx_start(f, f % _NBUF)
        for f in range(_PF):
            idx_wait(f, f % _NBUF)
            gather_start(f, f % _NBUF)

        def chunk(t, carry):
            b = t % _NBUF

            fi = t + _PFI

            @pl.when(fi < _NCHUNK)
            def _():
                b2 = fi % _NBUF

                @pl.when(fi >= _NBUF)
                def _():
                    scatter_wait(fi - _NBUF, b2)

                idx_start(fi, b2)

            f = t + _PF

            @pl.when(f < _NCHUNK)
            def _():
                bf = f % _NBUF
                idx_wait(f, bf)
                gather_start(f, bf)

            gather_wait(t, b)

            for g in range(_CH // 16):
                w16 = wbuf[b, pl.ds(g * 16, 16)]
                for l in range(16):
                    e = g * 16 + l
                    wsc = w16[l]
                    for q in range(D // 16):
                        sl = pl.ds(q * 16, 16)
                        rows_v[b, e, sl] = rows_v[b, e, sl] * wsc

            scatter_start(t, b)
            return carry

        lax.fori_loop(0, _NCHUNK, chunk, 0)
        for i in range(_NBUF):
            f = _NCHUNK - _NBUF + i
            scatter_wait(f, f % _NBUF)

        plsc.subcore_barrier()
        pltpu.sync_copy(acc.at[pl.ds(sid * _ZR, _ZR)],
                        out_hbm.at[cid, pl.ds(sid * _ZR, _ZR)])

    return spmm


def _mm1(x, W1):
    def body(x_ref, w_ref, o_ref):
        o_ref[...] = jnp.dot(x_ref[...], w_ref[...],
                             preferred_element_type=jnp.float32)

    return pl.pallas_call(
        body,
        grid=(_N // _BM,),
        in_specs=[pl.BlockSpec((_BM, _NFEAT), lambda i: (i, 0)),
                  pl.BlockSpec((_NFEAT, _NHID), lambda i: (0, 0))],
        out_specs=pl.BlockSpec((_BM, _NHID), lambda i: (i, 0)),
        out_shape=jax.ShapeDtypeStruct((_N, _NHID), jnp.float32),
    )(x, W1)


def _fuse1(parts, b1, W2):
    def body(p_ref, b_ref, w_ref, o_ref):
        h = jnp.maximum(p_ref[0] + p_ref[1] + b_ref[...], 0.0)
        o_ref[...] = jnp.dot(h, w_ref[...], preferred_element_type=jnp.float32)

    return pl.pallas_call(
        body,
        grid=(_N // _BM,),
        in_specs=[pl.BlockSpec((2, _BM, _NHID), lambda i: (0, i, 0)),
                  pl.BlockSpec((1, _NHID), lambda i: (0, 0)),
                  pl.BlockSpec((_NHID, _NHID2), lambda i: (0, 0))],
        out_specs=pl.BlockSpec((_BM, _NHID2), lambda i: (i, 0)),
        out_shape=jax.ShapeDtypeStruct((_N, _NHID2), jnp.float32),
    )(parts, b1.reshape(1, _NHID), W2)


def _fuse2(parts, b2):
    def body(p_ref, b_ref, o_ref):
        o_ref[...] = jnp.maximum(p_ref[0] + p_ref[1] + b_ref[...], 0.0)

    return pl.pallas_call(
        body,
        grid=(_N // _BM,),
        in_specs=[pl.BlockSpec((2, _BM, _NHID2), lambda i: (0, i, 0)),
                  pl.BlockSpec((1, _NHID2), lambda i: (0, 0))],
        out_specs=pl.BlockSpec((_BM, _NHID2), lambda i: (i, 0)),
        out_shape=jax.ShapeDtypeStruct((_N, _NHID2), jnp.float32),
    )(parts, b2.reshape(1, _NHID2))


def kernel(edge_index, edge_weight, emb_node, emb_attri, W1, b1, W2, b2):
    dst = edge_index[0].astype(jnp.int32).reshape(_NW, _NCHUNK, _CH)
    src = edge_index[1].astype(jnp.int32).reshape(_NW, _NCHUNK, _CH)
    w = edge_weight.astype(jnp.float32).reshape(_NW, _NCHUNK, _CH)
    edata = jnp.stack([src, dst], axis=2)

    x = jnp.concatenate([emb_node, emb_attri], axis=0)
    sup1 = _mm1(x, W1)
    part1 = _make_spmm(_NHID)(edata, w, sup1)
    sup2 = _fuse1(part1, b1, W2)
    part2 = _make_spmm(_NHID2)(edata, w, sup2)
    return _fuse2(part2, b2)


# speedup vs baseline: 1.2986x; 1.0675x over previous
import functools

import jax
import jax.numpy as jnp
from jax import lax
from jax.experimental import pallas as pl
from jax.experimental.pallas import tpu as pltpu
from jax.experimental.pallas import tpu_sc as plsc

_NNODE = 10000
_NATTRI = 6000
_NFEAT = 128
_NHID = 64
_NHID2 = 32
_E = 512000
_N = _NNODE + _NATTRI

_NW = 32
_CH = 128
_EPW = _E // _NW
_NCHUNK = _EPW // _CH
_ZR = _N // 16
_BM = 2000
_NBUF = 7
_PFI = 4
_PF = 2


def _make_spmm(D):
    mesh = plsc.VectorSubcoreMesh(core_axis_name="c", subcore_axis_name="s")

    @functools.partial(
        pl.kernel,
        out_type=jax.ShapeDtypeStruct((2, _N, D), jnp.float32),
        mesh=mesh,
        compiler_params=pltpu.CompilerParams(use_tc_tiling_on_sc=False),
        scratch_types=[
            pltpu.VMEM((_NBUF, 2, _CH), jnp.int32),
            pltpu.VMEM((_NBUF, _CH), jnp.float32),
            pltpu.VMEM((_NBUF, _CH, D), jnp.float32),
            pltpu.VMEM_SHARED((_N, D), jnp.float32),
            pltpu.SemaphoreType.DMA((_NBUF,)),
            pltpu.SemaphoreType.DMA((_NBUF,)),
            pltpu.SemaphoreType.DMA((_NBUF,)),
        ],
    )
    def spmm(edata_hbm, w_hbm, sup_hbm, out_hbm,
             ebuf, wbuf, rows_v, acc, isem, gsem, ssem):
        cid = lax.axis_index("c")
        sid = lax.axis_index("s")
        wid = sid * 2 + cid

        zvec = jnp.zeros((16,), jnp.float32)

        def zrow(e, carry):
            for q in range(D // 16):
                rows_v[0, e, pl.ds(q * 16, 16)] = zvec
            return carry

        lax.fori_loop(0, _CH, zrow, 0)
        for k in range(8):
            pltpu.sync_copy(rows_v.at[0].at[pl.ds(0, _ZR // 8)],
                            acc.at[pl.ds(sid * _ZR + k * (_ZR // 8), _ZR // 8)])
        plsc.subcore_barrier()

        def idx_start(f, bf):
            pltpu.async_copy(edata_hbm.at[wid, f], ebuf.at[bf], isem.at[bf])
            pltpu.async_copy(w_hbm.at[wid, f], wbuf.at[bf], isem.at[bf])

        def idx_wait(f, bf):
            pltpu.make_async_copy(edata_hbm.at[wid, f], ebuf.at[bf],
                                  isem.at[bf]).wait()
            pltpu.make_async_copy(w_hbm.at[wid, f], wbuf.at[bf],
                                  isem.at[bf]).wait()

        def gather_start(f, bf):
            pltpu.async_copy(sup_hbm.at[ebuf.at[bf, 0]], rows_v.at[bf],
                             gsem.at[bf])

        def gather_wait(f, bf):
            pltpu.make_async_copy(sup_hbm.at[ebuf.at[bf, 0]], rows_v.at[bf],
                                  gsem.at[bf]).wait()

        def scatter_start(f, bf):
            pltpu.async_copy(rows_v.at[bf], acc.at[ebuf.at[bf, 1]],
                             ssem.at[bf], add=True)

        def scatter_wait(f, bf):
            pltpu.make_async_copy(rows_v.at[bf], acc.at[ebuf.at[bf, 1]],
                                  ssem.at[bf]).wait()

        for f in range(_PFI):
            idx_start(f, f % _NBUF)
        for f in range(_PF):
            idx_wait(f, f % _NBUF)
            gather_start(f, f % _NBUF)

        def chunk(t, carry):
            b = t % _NBUF

            fi = t + _PFI

            @pl.when(fi < _NCHUNK)
            def _():
                b2 = fi % _NBUF

                @pl.when(fi >= _NBUF)
                def _():
                    scatter_wait(fi - _NBUF, b2)

                idx_start(fi, b2)

            f = t + _PF

            @pl.when(f < _NCHUNK)
            def _():
                bf = f % _NBUF
                idx_wait(f, bf)
                gather_start(f, bf)

            gather_wait(t, b)

            for g in range(_CH // 16):
                w16 = wbuf[b, pl.ds(g * 16, 16)]
                for l in range(16):
                    e = g * 16 + l
                    wsc = w16[l]
                    for q in range(D // 16):
                        sl = pl.ds(q * 16, 16)
                        rows_v[b, e, sl] = rows_v[b, e, sl] * wsc

            scatter_start(t, b)
            return carry

        lax.fori_loop(0, _NCHUNK, chunk, 0)
        for i in range(_NBUF):
            f = _NCHUNK - _NBUF + i
            scatter_wait(f, f % _NBUF)

        plsc.subcore_barrier()
        pltpu.sync_copy(acc.at[pl.ds(sid * _ZR, _ZR)],
                        out_hbm.at[cid, pl.ds(sid * _ZR, _ZR)])

    return spmm


def _mm1(x, W1):
    def body(x_ref, w_ref, o_ref):
        o_ref[...] = jnp.dot(x_ref[...], w_ref[...],
                             preferred_element_type=jnp.float32)

    return pl.pallas_call(
        body,
        grid=(_N // _BM,),
        in_specs=[pl.BlockSpec((_BM, _NFEAT), lambda i: (i, 0)),
                  pl.BlockSpec((_NFEAT, _NHID), lambda i: (0, 0))],
        out_specs=pl.BlockSpec((_BM, _NHID), lambda i: (i, 0)),
        out_shape=jax.ShapeDtypeStruct((_N, _NHID), jnp.float32),
    )(x, W1)


def _fuse1(parts, b1, W2):
    def body(p_ref, b_ref, w_ref, o_ref):
        h = jnp.maximum(p_ref[0] + p_ref[1] + b_ref[...], 0.0)
        o_ref[...] = jnp.dot(h, w_ref[...], preferred_element_type=jnp.float32)

    return pl.pallas_call(
        body,
        grid=(_N // _BM,),
        in_specs=[pl.BlockSpec((2, _BM, _NHID), lambda i: (0, i, 0)),
                  pl.BlockSpec((1, _NHID), lambda i: (0, 0)),
                  pl.BlockSpec((_NHID, _NHID2), lambda i: (0, 0))],
        out_specs=pl.BlockSpec((_BM, _NHID2), lambda i: (i, 0)),
        out_shape=jax.ShapeDtypeStruct((_N, _NHID2), jnp.float32),
    )(parts, b1.reshape(1, _NHID), W2)


def _fuse2(parts, b2):
    def body(p_ref, b_ref, o_ref):
        o_ref[...] = jnp.maximum(p_ref[0] + p_ref[1] + b_ref[...], 0.0)

    return pl.pallas_call(
        body,
        grid=(_N // _BM,),
        in_specs=[pl.BlockSpec((2, _BM, _NHID2), lambda i: (0, i, 0)),
                  pl.BlockSpec((1, _NHID2), lambda i: (0, 0))],
        out_specs=pl.BlockSpec((_BM, _NHID2), lambda i: (i, 0)),
        out_shape=jax.ShapeDtypeStruct((_N, _NHID2), jnp.float32),
    )(parts, b2.reshape(1, _NHID2))


def kernel(edge_index, edge_weight, emb_node, emb_attri, W1, b1, W2, b2):
    dst = edge_index[0].astype(jnp.int32).reshape(_NW, _NCHUNK, _CH)
    src = edge_index[1].astype(jnp.int32).reshape(_NW, _NCHUNK, _CH)
    w = edge_weight.astype(jnp.float32).reshape(_NW, _NCHUNK, _CH)
    edata = jnp.stack([src, dst], axis=2)

    x = jnp.concatenate([emb_node, emb_attri], axis=0)
    sup1 = _mm1(x, W1)
    part1 = _make_spmm(_NHID)(edata, w, sup1)
    sup2 = _fuse1(part1, b1, W2)
    part2 = _make_spmm(_NHID2)(edata, w, sup2)
    return _fuse2(part2, b2)


# speedup vs baseline: 1.2999x; 1.0010x over previous
import functools

import jax
import jax.numpy as jnp
from jax import lax
from jax.experimental import pallas as pl
from jax.experimental.pallas import tpu as pltpu
from jax.experimental.pallas import tpu_sc as plsc

_NNODE = 10000
_NATTRI = 6000
_NFEAT = 128
_NHID = 64
_NHID2 = 32
_E = 512000
_N = _NNODE + _NATTRI

_NW = 32
_CH = 128
_EPW = _E // _NW
_NCHUNK = _EPW // _CH
_ZR = _N // 16
_BM = 2000
_PFI = 4
_PF = 2


def _make_spmm(D):
    mesh = plsc.VectorSubcoreMesh(core_axis_name="c", subcore_axis_name="s")
    _NBUF = 7 if D == 64 else 10

    @functools.partial(
        pl.kernel,
        out_type=jax.ShapeDtypeStruct((2, _N, D), jnp.float32),
        mesh=mesh,
        compiler_params=pltpu.CompilerParams(use_tc_tiling_on_sc=False),
        scratch_types=[
            pltpu.VMEM((_NBUF, 2, _CH), jnp.int32),
            pltpu.VMEM((_NBUF, _CH), jnp.float32),
            pltpu.VMEM((_NBUF, _CH, D), jnp.float32),
            pltpu.VMEM_SHARED((_N, D), jnp.float32),
            pltpu.SemaphoreType.DMA((_NBUF,)),
            pltpu.SemaphoreType.DMA((_NBUF,)),
            pltpu.SemaphoreType.DMA((_NBUF,)),
        ],
    )
    def spmm(edata_hbm, w_hbm, sup_hbm, out_hbm,
             ebuf, wbuf, rows_v, acc, isem, gsem, ssem):
        cid = lax.axis_index("c")
        sid = lax.axis_index("s")
        wid = sid * 2 + cid

        zvec = jnp.zeros((16,), jnp.float32)

        def zrow(e, carry):
            for q in range(D // 16):
                rows_v[0, e, pl.ds(q * 16, 16)] = zvec
            return carry

        lax.fori_loop(0, _CH, zrow, 0)
        for k in range(8):
            pltpu.sync_copy(rows_v.at[0].at[pl.ds(0, _ZR // 8)],
                            acc.at[pl.ds(sid * _ZR + k * (_ZR // 8), _ZR // 8)])
        plsc.subcore_barrier()

        def idx_start(f, bf):
            pltpu.async_copy(edata_hbm.at[wid, f], ebuf.at[bf], isem.at[bf])
            pltpu.async_copy(w_hbm.at[wid, f], wbuf.at[bf], isem.at[bf])

        def idx_wait(f, bf):
            pltpu.make_async_copy(edata_hbm.at[wid, f], ebuf.at[bf],
                                  isem.at[bf]).wait()
            pltpu.make_async_copy(w_hbm.at[wid, f], wbuf.at[bf],
                                  isem.at[bf]).wait()

        def gather_start(f, bf):
            pltpu.async_copy(sup_hbm.at[ebuf.at[bf, 0]], rows_v.at[bf],
                             gsem.at[bf])

        def gather_wait(f, bf):
            pltpu.make_async_copy(sup_hbm.at[ebuf.at[bf, 0]], rows_v.at[bf],
                                  gsem.at[bf]).wait()

        def scatter_start(f, bf):
            pltpu.async_copy(rows_v.at[bf], acc.at[ebuf.at[bf, 1]],
                             ssem.at[bf], add=True)

        def scatter_wait(f, bf):
            pltpu.make_async_copy(rows_v.at[bf], acc.at[ebuf.at[bf, 1]],
                                  ssem.at[bf]).wait()

        for f in range(_PFI):
            idx_start(f, f % _NBUF)
        for f in range(_PF):
            idx_wait(f, f % _NBUF)
            gather_start(f, f % _NBUF)

        def chunk(t, carry):
            b = t % _NBUF

            fi = t + _PFI

            @pl.when(fi < _NCHUNK)
            def _():
                b2 = fi % _NBUF

                @pl.when(fi >= _NBUF)
                def _():
                    scatter_wait(fi - _NBUF, b2)

                idx_start(fi, b2)

            f = t + _PF

            @pl.when(f < _NCHUNK)
            def _():
                bf = f % _NBUF
                idx_wait(f, bf)
                gather_start(f, bf)

            gather_wait(t, b)

            for g in range(_CH // 16):
                w16 = wbuf[b, pl.ds(g * 16, 16)]
                for l in range(16):
                    e = g * 16 + l
                    wsc = w16[l]
                    for q in range(D // 16):
                        sl = pl.ds(q * 16, 16)
                        rows_v[b, e, sl] = rows_v[b, e, sl] * wsc

            scatter_start(t, b)
            return carry

        lax.fori_loop(0, _NCHUNK, chunk, 0)
        for i in range(_NBUF):
            f = _NCHUNK - _NBUF + i
            scatter_wait(f, f % _NBUF)

        plsc.subcore_barrier()
        pltpu.sync_copy(acc.at[pl.ds(sid * _ZR, _ZR)],
                        out_hbm.at[cid, pl.ds(sid * _ZR, _ZR)])

    return spmm


def _mm1(x, W1):
    def body(x_ref, w_ref, o_ref):
        o_ref[...] = jnp.dot(x_ref[...], w_ref[...],
                             preferred_element_type=jnp.float32)

    return pl.pallas_call(
        body,
        grid=(_N // _BM,),
        in_specs=[pl.BlockSpec((_BM, _NFEAT), lambda i: (i, 0)),
                  pl.BlockSpec((_NFEAT, _NHID), lambda i: (0, 0))],
        out_specs=pl.BlockSpec((_BM, _NHID), lambda i: (i, 0)),
        out_shape=jax.ShapeDtypeStruct((_N, _NHID), jnp.float32),
    )(x, W1)


def _fuse1(parts, b1, W2):
    def body(p_ref, b_ref, w_ref, o_ref):
        h = jnp.maximum(p_ref[0] + p_ref[1] + b_ref[...], 0.0)
        o_ref[...] = jnp.dot(h, w_ref[...], preferred_element_type=jnp.float32)

    return pl.pallas_call(
        body,
        grid=(_N // _BM,),
        in_specs=[pl.BlockSpec((2, _BM, _NHID), lambda i: (0, i, 0)),
                  pl.BlockSpec((1, _NHID), lambda i: (0, 0)),
                  pl.BlockSpec((_NHID, _NHID2), lambda i: (0, 0))],
        out_specs=pl.BlockSpec((_BM, _NHID2), lambda i: (i, 0)),
        out_shape=jax.ShapeDtypeStruct((_N, _NHID2), jnp.float32),
    )(parts, b1.reshape(1, _NHID), W2)


def _fuse2(parts, b2):
    def body(p_ref, b_ref, o_ref):
        o_ref[...] = jnp.maximum(p_ref[0] + p_ref[1] + b_ref[...], 0.0)

    return pl.pallas_call(
        body,
        grid=(_N // _BM,),
        in_specs=[pl.BlockSpec((2, _BM, _NHID2), lambda i: (0, i, 0)),
                  pl.BlockSpec((1, _NHID2), lambda i: (0, 0))],
        out_specs=pl.BlockSpec((_BM, _NHID2), lambda i: (i, 0)),
        out_shape=jax.ShapeDtypeStruct((_N, _NHID2), jnp.float32),
    )(parts, b2.reshape(1, _NHID2))


def kernel(edge_index, edge_weight, emb_node, emb_attri, W1, b1, W2, b2):
    dst = edge_index[0].astype(jnp.int32).reshape(_NW, _NCHUNK, _CH)
    src = edge_index[1].astype(jnp.int32).reshape(_NW, _NCHUNK, _CH)
    w = edge_weight.astype(jnp.float32).reshape(_NW, _NCHUNK, _CH)
    edata = jnp.stack([src, dst], axis=2)

    x = jnp.concatenate([emb_node, emb_attri], axis=0)
    sup1 = _mm1(x, W1)
    part1 = _make_spmm(_NHID)(edata, w, sup1)
    sup2 = _fuse1(part1, b1, W2)
    part2 = _make_spmm(_NHID2)(edata, w, sup2)
    return _fuse2(part2, b2)
